# baseline (speedup 1.0000x reference)
import jax
import jax.numpy as jnp
from jax import lax
from jax.experimental import pallas as pl
from jax.experimental.pallas import tpu as pltpu

B, H, W, NSRC = 8, 2048, 2048, 128
STRIDE = H // NSRC
NCHUNK = 16


def _fast_body(yr, yc, xc, out, plane, sem_big, sem_in, sem_out):
    plane_in = pltpu.make_async_copy(yr.at[:, :, 0, :], plane, sem_in)
    plane_in.start()
    copies = [
        pltpu.make_async_copy(
            yr.at[pl.ds(i % B, 1), pl.ds((i // B) * (NSRC // (NCHUNK // B)),
                                         NSRC // (NCHUNK // B))],
            out.at[pl.ds(i % B, 1), pl.ds((i // B) * (NSRC // (NCHUNK // B)),
                                          NSRC // (NCHUNK // B))],
            sem_big,
        )
        for i in range(NCHUNK)
    ]
    for c in copies:
        c.start()
    plane_in.wait()
    col = lax.broadcasted_iota(jnp.int32, (B, NSRC, W), 2)
    upd = jnp.where(col == yc[...][None, :, :], xc[...][None, :, :], 0.0)
    plane[...] = plane[...] + upd
    for c in copies:
        c.wait()
    plane_out = pltpu.make_async_copy(plane, out.at[:, :, 0, :], sem_out)
    plane_out.start()
    plane_out.wait()


def _fast(Y, X, x, y):
    Yr = Y.reshape(B, NSRC, STRIDE, W)
    yc = y.reshape(NSRC, 1)
    Xc = X.reshape(NSRC, 1)
    out = pl.pallas_call(
        _fast_body,
        in_specs=[
            pl.BlockSpec(memory_space=pl.ANY),
            pl.BlockSpec(memory_space=pltpu.VMEM),
            pl.BlockSpec(memory_space=pltpu.VMEM),
        ],
        out_specs=pl.BlockSpec(memory_space=pl.ANY),
        out_shape=jax.ShapeDtypeStruct((B, NSRC, STRIDE, W), jnp.float32),
        scratch_shapes=[
            pltpu.VMEM((B, NSRC, W), jnp.float32),
            pltpu.SemaphoreType.DMA,
            pltpu.SemaphoreType.DMA,
            pltpu.SemaphoreType.DMA,
        ],
        input_output_aliases={0: 0},
    )(Yr, yc, Xc)
    return out.reshape(B, H, W)


FR = 1024
NBLK = (B * H) // FR


def _gen_body(lo_ref, hi_ref, xf_ref, yf_ref, xvf_ref, yin, yout):
    g = pl.program_id(0)
    yout[...] = yin[...]
    r0 = g * FR

    def upd(i, carry):
        dr = xf_ref[i] - r0
        yi = yf_ref[i]
        xv = xvf_ref[i]
        col = lax.broadcasted_iota(jnp.int32, (1, W), 1)
        row = yout[pl.ds(dr, 1), :]
        yout[pl.ds(dr, 1), :] = row + jnp.where(col == yi, xv, 0.0)
        return carry

    lax.fori_loop(lo_ref[g], hi_ref[g], upd, 0)


def _generic(Y, X, x, y):
    Yf = Y.reshape(B * H, W)
    xf = (jnp.arange(B, dtype=jnp.int32)[:, None] * H + x[None, :]).reshape(-1)
    yf = jnp.broadcast_to(y, (B, NSRC)).reshape(-1)
    xvf = jnp.broadcast_to(X, (B, NSRC)).reshape(-1)

    block_starts = jnp.arange(NBLK, dtype=jnp.int32) * FR
    lo = jnp.searchsorted(xf, block_starts, side="left").astype(jnp.int32)
    hi = jnp.searchsorted(xf, block_starts + FR, side="left").astype(jnp.int32)

    grid_spec = pltpu.PrefetchScalarGridSpec(
        num_scalar_prefetch=5,
        grid=(NBLK,),
        in_specs=[pl.BlockSpec((FR, W), lambda g, *refs: (g, 0))],
        out_specs=pl.BlockSpec((FR, W), lambda g, *refs: (g, 0)),
    )
    out = pl.pallas_call(
        _gen_body,
        grid_spec=grid_spec,
        out_shape=jax.ShapeDtypeStruct((B * H, W), jnp.float32),
    )(lo, hi, xf, yf, xvf, Yf)
    return out.reshape(B, H, W)


def kernel(Y, X, x, y):
    structured = jnp.all(x == jnp.arange(NSRC, dtype=jnp.int32) * STRIDE)
    return lax.cond(structured, _fast, _generic, Y, X, x, y)


# speedup vs baseline: 44.0293x; 44.0293x over previous
import jax
import jax.numpy as jnp
from jax import lax
from jax.experimental import pallas as pl
from jax.experimental.pallas import tpu as pltpu

B, H, W, NSRC = 8, 2048, 2048, 128
STRIDE = H // NSRC
NCHUNK = 16


def _fast_body(yr, yc, xc, out, plane, sem_in, sem_out):
    plane_in = pltpu.make_async_copy(yr.at[:, :, 0, :], plane, sem_in)
    plane_in.start()
    plane_in.wait()
    col = lax.broadcasted_iota(jnp.int32, (B, NSRC, W), 2)
    upd = jnp.where(col == yc[...][None, :, :], xc[...][None, :, :], 0.0)
    plane[...] = plane[...] + upd
    plane_out = pltpu.make_async_copy(plane, out.at[:, :, 0, :], sem_out)
    plane_out.start()
    plane_out.wait()


def _fast(Y, X, x, y):
    Yr = Y.reshape(B, NSRC, STRIDE, W)
    yc = y.reshape(NSRC, 1)
    Xc = X.reshape(NSRC, 1)
    out = pl.pallas_call(
        _fast_body,
        in_specs=[
            pl.BlockSpec(memory_space=pl.ANY),
            pl.BlockSpec(memory_space=pltpu.VMEM),
            pl.BlockSpec(memory_space=pltpu.VMEM),
        ],
        out_specs=pl.BlockSpec(memory_space=pl.ANY),
        out_shape=jax.ShapeDtypeStruct((B, NSRC, STRIDE, W), jnp.float32),
        scratch_shapes=[
            pltpu.VMEM((B, NSRC, W), jnp.float32),
            pltpu.SemaphoreType.DMA,
            pltpu.SemaphoreType.DMA,
        ],
        input_output_aliases={0: 0},
    )(Yr, yc, Xc)
    return out.reshape(B, H, W)


FR = 1024
NBLK = (B * H) // FR


def _gen_body(lo_ref, hi_ref, xf_ref, yf_ref, xvf_ref, yin, yout):
    g = pl.program_id(0)
    yout[...] = yin[...]
    r0 = g * FR

    def upd(i, carry):
        dr = xf_ref[i] - r0
        yi = yf_ref[i]
        xv = xvf_ref[i]
        col = lax.broadcasted_iota(jnp.int32, (1, W), 1)
        row = yout[pl.ds(dr, 1), :]
        yout[pl.ds(dr, 1), :] = row + jnp.where(col == yi, xv, 0.0)
        return carry

    lax.fori_loop(lo_ref[g], hi_ref[g], upd, 0)


def _generic(Y, X, x, y):
    Yf = Y.reshape(B * H, W)
    xf = (jnp.arange(B, dtype=jnp.int32)[:, None] * H + x[None, :]).reshape(-1)
    yf = jnp.broadcast_to(y, (B, NSRC)).reshape(-1)
    xvf = jnp.broadcast_to(X, (B, NSRC)).reshape(-1)

    block_starts = jnp.arange(NBLK, dtype=jnp.int32) * FR
    lo = jnp.searchsorted(xf, block_starts, side="left").astype(jnp.int32)
    hi = jnp.searchsorted(xf, block_starts + FR, side="left").astype(jnp.int32)

    grid_spec = pltpu.PrefetchScalarGridSpec(
        num_scalar_prefetch=5,
        grid=(NBLK,),
        in_specs=[pl.BlockSpec((FR, W), lambda g, *refs: (g, 0))],
        out_specs=pl.BlockSpec((FR, W), lambda g, *refs: (g, 0)),
    )
    out = pl.pallas_call(
        _gen_body,
        grid_spec=grid_spec,
        out_shape=jax.ShapeDtypeStruct((B * H, W), jnp.float32),
    )(lo, hi, xf, yf, xvf, Yf)
    return out.reshape(B, H, W)


def kernel(Y, X, x, y):
    structured = jnp.all(x == jnp.arange(NSRC, dtype=jnp.int32) * STRIDE)
    return lax.cond(structured, _fast, _generic, Y, X, x, y)
